# Initial kernel scaffold; baseline (speedup 1.0000x reference)
import jax
import jax.numpy as jnp
from jax.experimental import pallas as pl


def kernel(x, table):
    raise NotImplementedError("write your pallas kernel here")


# speedup vs baseline: 1.4373x; 1.4373x over previous
import functools

import jax
import jax.numpy as jnp
from jax import lax
from jax.experimental import pallas as pl
from jax.experimental.pallas import tpu as pltpu
from jax.experimental.pallas import tpu_sc as plsc

_DIM2 = 32
_BATCH = 16384
_FIELDS = 26
_B = _BATCH * _FIELDS

_info = plsc.get_sparse_core_info()
_NC = _info.num_cores
_NS = _info.num_subcores
_NW = _NC * _NS
_B_PER_W = _B // _NW
_CHUNK = 128
_N_CHUNKS = _B_PER_W // _CHUNK


@functools.partial(
    pl.kernel,
    mesh=plsc.VectorSubcoreMesh(core_axis_name="c", subcore_axis_name="s"),
    out_type=jax.ShapeDtypeStruct((_NW, _B_PER_W, _DIM2), jnp.float32),
    scratch_types=[
        pltpu.VMEM((_N_CHUNKS, _CHUNK), jnp.int32),
        pltpu.VMEM((_CHUNK, _DIM2), jnp.float32),
        pltpu.SemaphoreType.DMA,
    ],
    compiler_params=pltpu.CompilerParams(use_tc_tiling_on_sc=False),
)
def _gather_sc(x_hbm, table_hbm, out_hbm, idx_v, rows_v, sem):
    wid = lax.axis_index("s") * _NC + lax.axis_index("c")
    pltpu.sync_copy(x_hbm.at[wid], idx_v)

    def step(j, carry):
        pltpu.async_copy(table_hbm.at[idx_v.at[j]], rows_v, sem).wait()
        pltpu.sync_copy(rows_v, out_hbm.at[wid, pl.ds(j * _CHUNK, _CHUNK)])
        return carry

    lax.fori_loop(0, _N_CHUNKS, step, 0)


def kernel(x, table):
    xw = x.reshape(_NW, _N_CHUNKS, _CHUNK)
    out = _gather_sc(xw, table)
    return out.reshape(_BATCH, _FIELDS, _DIM2)


# speedup vs baseline: 1.5674x; 1.0906x over previous
import functools

import jax
import jax.numpy as jnp
from jax import lax
from jax.experimental import pallas as pl
from jax.experimental.pallas import tpu as pltpu
from jax.experimental.pallas import tpu_sc as plsc

_DIM2 = 32
_BATCH = 16384
_FIELDS = 26
_B = _BATCH * _FIELDS

_info = plsc.get_sparse_core_info()
_NC = _info.num_cores
_NS = _info.num_subcores
_NW = _NC * _NS
_B_PER_W = _B // _NW
_CHUNK = 128
_N_CHUNKS = _B_PER_W // _CHUNK
_K = 13
_GROUP = _K * _CHUNK
_NG = _N_CHUNKS // _K


@functools.partial(
    pl.kernel,
    mesh=plsc.VectorSubcoreMesh(core_axis_name="c", subcore_axis_name="s"),
    out_type=jax.ShapeDtypeStruct((_NW, _B_PER_W, _DIM2), jnp.float32),
    scratch_types=[
        pltpu.VMEM((_N_CHUNKS, _CHUNK), jnp.int32),
        pltpu.VMEM((_GROUP, _DIM2), jnp.float32),
        pltpu.VMEM((_GROUP, _DIM2), jnp.float32),
        pltpu.SemaphoreType.DMA,
        pltpu.SemaphoreType.DMA,
        pltpu.SemaphoreType.DMA,
        pltpu.SemaphoreType.DMA,
    ],
    compiler_params=pltpu.CompilerParams(use_tc_tiling_on_sc=False),
)
def _gather_sc(x_hbm, table_hbm, out_hbm, idx_v, buf0, buf1, g0, g1, w0, w1):
    wid = lax.axis_index("s") * _NC + lax.axis_index("c")
    pltpu.sync_copy(x_hbm.at[wid], idx_v)
    bufs, gsems, wsems = (buf0, buf1), (g0, g1), (w0, w1)

    def wait_write(b):
        pltpu.make_async_copy(
            bufs[b], out_hbm.at[wid, pl.ds(0, _GROUP)], wsems[b]).wait()

    def do_group(gg, b, first):
        if not first:
            wait_write(b)
        handles = [
            pltpu.async_copy(
                table_hbm.at[idx_v.at[gg * _K + k]],
                bufs[b].at[pl.ds(k * _CHUNK, _CHUNK)],
                gsems[b])
            for k in range(_K)
        ]
        for h in handles:
            h.wait()
        pltpu.async_copy(
            bufs[b], out_hbm.at[wid, pl.ds(gg * _GROUP, _GROUP)], wsems[b])

    do_group(0, 0, True)
    do_group(1, 1, True)

    @pl.loop(0, _NG - 2, step=2)
    def _groups(g):
        do_group(g + 2, 0, False)
        do_group(g + 3, 1, False)

    wait_write(0)
    wait_write(1)


def kernel(x, table):
    xw = x.reshape(_NW, _N_CHUNKS, _CHUNK)
    out = _gather_sc(xw, table)
    return out.reshape(_BATCH, _FIELDS, _DIM2)
